# baseline (speedup 1.0000x reference)
import jax
import jax.numpy as jnp
from jax import lax
from jax.experimental import pallas as pl
from jax.experimental.pallas import tpu as pltpu
from jax.experimental.pallas import tpu_sc as plsc

N = 10000
E = 320000
D = 128
EPS = 1e-5

NC = 2
NS = 16
NW = NC * NS
CHUNK = 128
NCH = 80
HCH = NCH // 2
E_PAD = NW * NCH * CHUNK
NZ = N + 8
R0 = 624
RLAST = N - (NS - 1) * R0


def _segsum_body(src_hbm, dst_hbm, z_hbm, zeros_hbm, out_hbm,
                 sa, da, sb, db, rows_a, rows_b, sem_a, sem_b, agg):
    c = lax.axis_index("c")
    s = lax.axis_index("s")
    w = c * NS + s
    base = pl.multiple_of(w * NCH * CHUNK, 8)
    row0 = pl.multiple_of(s * R0, 8)

    @pl.when(s < NS - 1)
    def _():
        pltpu.sync_copy(zeros_hbm.at[pl.ds(0, R0)], agg.at[pl.ds(row0, R0)])

    @pl.when(s == NS - 1)
    def _():
        pltpu.sync_copy(zeros_hbm, agg.at[pl.ds((NS - 1) * R0, RLAST)])

    plsc.subcore_barrier()

    def body(i, carry):
        off = base + i * CHUNK
        pltpu.sync_copy(src_hbm.at[pl.ds(off, CHUNK)], sa)
        pltpu.sync_copy(dst_hbm.at[pl.ds(off, CHUNK)], da)
        pltpu.async_copy(z_hbm.at[sa], rows_a, sem_a).wait()
        pltpu.sync_copy(rows_a, agg.at[da], add=True)
        return carry

    lax.fori_loop(0, NCH, body, 0)

    plsc.subcore_barrier()
    obase = pl.multiple_of(c * N + row0, 8)

    @pl.when(s < NS - 1)
    def _():
        pltpu.sync_copy(agg.at[pl.ds(row0, R0)], out_hbm.at[pl.ds(obase, R0)])

    @pl.when(s == NS - 1)
    def _():
        pltpu.sync_copy(agg.at[pl.ds((NS - 1) * R0, RLAST)],
                        out_hbm.at[pl.ds(c * N + (NS - 1) * R0, RLAST)])


def _segment_sum(z, src, dst, zeros):
    mesh = plsc.VectorSubcoreMesh(core_axis_name="c", subcore_axis_name="s")
    k = pl.kernel(
        _segsum_body,
        mesh=mesh,
        out_type=jax.ShapeDtypeStruct((2 * N, D), jnp.float32),
        scratch_types=[
            pltpu.VMEM((CHUNK,), jnp.int32),
            pltpu.VMEM((CHUNK,), jnp.int32),
            pltpu.VMEM((CHUNK,), jnp.int32),
            pltpu.VMEM((CHUNK,), jnp.int32),
            pltpu.VMEM((CHUNK, D), jnp.float32),
            pltpu.VMEM((CHUNK, D), jnp.float32),
            pltpu.SemaphoreType.DMA,
            pltpu.SemaphoreType.DMA,
            pltpu.VMEM_SHARED((N, D), jnp.float32),
        ],
    )
    return k(src, dst, z, zeros)


BM = 1000


def _mlp_body(x_ref, p0_ref, p1_ref, w1_ref, b1_ref, w2_ref, b2_ref, o_ref):
    h = x_ref[...] + p0_ref[...] + p1_ref[...]
    h = jnp.dot(h, w1_ref[...], preferred_element_type=jnp.float32) + b1_ref[...]
    h = jnp.maximum(h, 0.0)
    h = jnp.dot(h, w2_ref[...], preferred_element_type=jnp.float32) + b2_ref[...]
    o_ref[...] = jnp.maximum(h, 0.0)


def _gin_mlp(x, parts, w1, b1, w2, b2):
    nb = N // BM
    return pl.pallas_call(
        _mlp_body,
        grid=(nb,),
        in_specs=[
            pl.BlockSpec((BM, D), lambda i: (i, 0)),
            pl.BlockSpec((BM, D), lambda i: (i, 0)),
            pl.BlockSpec((BM, D), lambda i, nb=nb: (i + nb, 0)),
            pl.BlockSpec((D, D), lambda i: (0, 0)),
            pl.BlockSpec((1, D), lambda i: (0, 0)),
            pl.BlockSpec((D, D), lambda i: (0, 0)),
            pl.BlockSpec((1, D), lambda i: (0, 0)),
        ],
        out_specs=pl.BlockSpec((BM, D), lambda i: (i, 0)),
        out_shape=jax.ShapeDtypeStruct((N, D), jnp.float32),
    )(x, parts, parts, w1, b1.reshape(1, D), w2, b2.reshape(1, D))


def _final_body(z2_ref, wp_ref, bp_ref, bng_ref, bnb_ref, png_ref, pnb_ref,
                pw_ref, z_ref, p_ref):
    z2 = z2_ref[...]
    m = jnp.mean(z2, axis=0, keepdims=True)
    v = jnp.mean((z2 - m) ** 2, axis=0, keepdims=True)
    z = (z2 - m) / jnp.sqrt(v + EPS) * bng_ref[...] + bnb_ref[...]
    z_ref[...] = z
    pp = jnp.dot(z, wp_ref[...], preferred_element_type=jnp.float32) + bp_ref[...]
    m2 = jnp.mean(pp, axis=0, keepdims=True)
    v2 = jnp.mean((pp - m2) ** 2, axis=0, keepdims=True)
    p = (pp - m2) / jnp.sqrt(v2 + EPS) * png_ref[...] + pnb_ref[...]
    p_ref[...] = jnp.where(p >= 0.0, p, pw_ref[0, 0] * p)


def _final(z2, wp, bp, bn_g, bn_b, pn_g, pn_b, prelu_w):
    return pl.pallas_call(
        _final_body,
        out_shape=(
            jax.ShapeDtypeStruct((N, D), jnp.float32),
            jax.ShapeDtypeStruct((N, D), jnp.float32),
        ),
    )(z2, wp, bp.reshape(1, D), bn_g.reshape(1, D), bn_b.reshape(1, D),
      pn_g.reshape(1, D), pn_b.reshape(1, D), prelu_w.reshape(1, 1))


def kernel(x, edge_index, W1_0, b1_0, W2_0, b2_0, W1_1, b1_1, W2_1, b2_1,
           bn_g, bn_b, Wp, bp, pn_g, pn_b, prelu_w):
    pad_s = jnp.full((E_PAD - E,), N, jnp.int32)
    pad_d = jnp.zeros((E_PAD - E,), jnp.int32)
    src = jnp.concatenate([edge_index[0], pad_s])
    dst = jnp.concatenate([edge_index[1], pad_d])
    zeros = jnp.zeros((RLAST, D), jnp.float32)
    zrow = jnp.zeros((NZ - N, D), jnp.float32)
    parts0 = _segment_sum(jnp.concatenate([x, zrow]), src, dst, zeros)
    z1 = _gin_mlp(x, parts0, W1_0, b1_0, W2_0, b2_0)
    parts1 = _segment_sum(jnp.concatenate([z1, zrow]), src, dst, zeros)
    z2 = _gin_mlp(z1, parts1, W1_1, b1_1, W2_1, b2_1)
    z, p = _final(z2, Wp, bp, bn_g, bn_b, pn_g, pn_b, prelu_w)
    return (z, p)


# speedup vs baseline: 1.0011x; 1.0011x over previous
import jax
import jax.numpy as jnp
from jax import lax
from jax.experimental import pallas as pl
from jax.experimental.pallas import tpu as pltpu
from jax.experimental.pallas import tpu_sc as plsc

N = 10000
E = 320000
D = 128
EPS = 1e-5

NC = 2
NS = 16
NW = NC * NS
CHUNK = 128
NCH = 80
HCH = NCH // 2
E_PAD = NW * NCH * CHUNK
NZ = N + 8
R0 = 624
RLAST = N - (NS - 1) * R0


def _segsum_body(src_hbm, dst_hbm, z_hbm, zeros_hbm, out_hbm,
                 sa, da, sb, db, rows_a, rows_b, sem_a, sem_b, agg):
    c = lax.axis_index("c")
    s = lax.axis_index("s")
    w = c * NS + s
    base = pl.multiple_of(w * NCH * CHUNK, 8)
    row0 = pl.multiple_of(s * R0, 8)

    @pl.when(s < NS - 1)
    def _():
        pltpu.sync_copy(zeros_hbm.at[pl.ds(0, R0)], agg.at[pl.ds(row0, R0)])

    @pl.when(s == NS - 1)
    def _():
        pltpu.sync_copy(zeros_hbm, agg.at[pl.ds((NS - 1) * R0, RLAST)])

    plsc.subcore_barrier()

    def body(i, carry):
        off = base + i * CHUNK
        pltpu.sync_copy(src_hbm.at[pl.ds(off, CHUNK)], sa)
        pltpu.sync_copy(dst_hbm.at[pl.ds(off, CHUNK)], da)
        pltpu.async_copy(z_hbm.at[sa], rows_a, sem_a).wait()
        pltpu.sync_copy(rows_a, agg.at[da], add=True)
        return carry

    lax.fori_loop(0, NCH, body, 0)

    plsc.subcore_barrier()
    obase = pl.multiple_of(c * N + row0, 8)

    @pl.when(s < NS - 1)
    def _():
        pltpu.sync_copy(agg.at[pl.ds(row0, R0)], out_hbm.at[pl.ds(obase, R0)])

    @pl.when(s == NS - 1)
    def _():
        pltpu.sync_copy(agg.at[pl.ds((NS - 1) * R0, RLAST)],
                        out_hbm.at[pl.ds(c * N + (NS - 1) * R0, RLAST)])


def _segment_sum(z, src, dst, zeros):
    mesh = plsc.VectorSubcoreMesh(core_axis_name="c", subcore_axis_name="s")
    k = pl.kernel(
        _segsum_body,
        mesh=mesh,
        out_type=jax.ShapeDtypeStruct((2 * N, D), jnp.float32),
        scratch_types=[
            pltpu.VMEM((CHUNK,), jnp.int32),
            pltpu.VMEM((CHUNK,), jnp.int32),
            pltpu.VMEM((CHUNK,), jnp.int32),
            pltpu.VMEM((CHUNK,), jnp.int32),
            pltpu.VMEM((CHUNK, D), jnp.float32),
            pltpu.VMEM((CHUNK, D), jnp.float32),
            pltpu.SemaphoreType.DMA,
            pltpu.SemaphoreType.DMA,
            pltpu.VMEM_SHARED((N, D), jnp.float32),
        ],
    )
    return k(src, dst, z, zeros)


BM = 1000


def _mlp_body(x_ref, p0_ref, p1_ref, w1_ref, b1_ref, w2_ref, b2_ref, o_ref):
    h = x_ref[...] + p0_ref[...] + p1_ref[...]
    h = jnp.dot(h, w1_ref[...], preferred_element_type=jnp.float32) + b1_ref[...]
    h = jnp.maximum(h, 0.0)
    h = jnp.dot(h, w2_ref[...], preferred_element_type=jnp.float32) + b2_ref[...]
    o_ref[...] = jnp.maximum(h, 0.0)


def _gin_mlp(x, parts, w1, b1, w2, b2):
    nb = N // BM
    return pl.pallas_call(
        _mlp_body,
        grid=(nb,),
        in_specs=[
            pl.BlockSpec((BM, D), lambda i: (i, 0)),
            pl.BlockSpec((BM, D), lambda i: (i, 0)),
            pl.BlockSpec((BM, D), lambda i, nb=nb: (i + nb, 0)),
            pl.BlockSpec((D, D), lambda i: (0, 0)),
            pl.BlockSpec((1, D), lambda i: (0, 0)),
            pl.BlockSpec((D, D), lambda i: (0, 0)),
            pl.BlockSpec((1, D), lambda i: (0, 0)),
        ],
        out_specs=pl.BlockSpec((BM, D), lambda i: (i, 0)),
        out_shape=jax.ShapeDtypeStruct((N, D), jnp.float32),
    )(x, parts, parts, w1, b1.reshape(1, D), w2, b2.reshape(1, D))


def _final_body(z2_ref, wp_ref, bp_ref, bng_ref, bnb_ref, png_ref, pnb_ref,
                pw_ref, z_ref, p_ref):
    z2 = z2_ref[...]
    m = jnp.mean(z2, axis=0, keepdims=True)
    v = jnp.mean((z2 - m) ** 2, axis=0, keepdims=True)
    z = (z2 - m) / jnp.sqrt(v + EPS) * bng_ref[...] + bnb_ref[...]
    z_ref[...] = z
    pp = jnp.dot(z, wp_ref[...], preferred_element_type=jnp.float32) + bp_ref[...]
    m2 = jnp.mean(pp, axis=0, keepdims=True)
    v2 = jnp.mean((pp - m2) ** 2, axis=0, keepdims=True)
    p = (pp - m2) / jnp.sqrt(v2 + EPS) * png_ref[...] + pnb_ref[...]
    p_ref[...] = jnp.where(p >= 0.0, p, pw_ref[0, 0] * p)


def _final(z2, wp, bp, bn_g, bn_b, pn_g, pn_b, prelu_w):
    return pl.pallas_call(
        _final_body,
        out_shape=(
            jax.ShapeDtypeStruct((N, D), jnp.float32),
            jax.ShapeDtypeStruct((N, D), jnp.float32),
        ),
    )(z2, wp, bp.reshape(1, D), bn_g.reshape(1, D), bn_b.reshape(1, D),
      pn_g.reshape(1, D), pn_b.reshape(1, D), prelu_w.reshape(1, 1))


def kernel(x, edge_index, W1_0, b1_0, W2_0, b2_0, W1_1, b1_1, W2_1, b2_1,
           bn_g, bn_b, Wp, bp, pn_g, pn_b, prelu_w):
    pad_s = jnp.full((E_PAD - E,), N, jnp.int32)
    pad_d = jnp.arange(E_PAD - E, dtype=jnp.int32) % N
    src = jnp.concatenate([edge_index[0], pad_s])
    dst = jnp.concatenate([edge_index[1], pad_d])
    zeros = jnp.zeros((RLAST, D), jnp.float32)
    zrow = jnp.zeros((NZ - N, D), jnp.float32)
    parts0 = _segment_sum(jnp.concatenate([x, zrow]), src, dst, zeros)
    z1 = _gin_mlp(x, parts0, W1_0, b1_0, W2_0, b2_0)
    parts1 = _segment_sum(jnp.concatenate([z1, zrow]), src, dst, zeros)
    z2 = _gin_mlp(z1, parts1, W1_1, b1_1, W2_1, b2_1)
    z, p = _final(z2, Wp, bp, bn_g, bn_b, pn_g, pn_b, prelu_w)
    return (z, p)


# speedup vs baseline: 1.9986x; 1.9963x over previous
import jax
import jax.numpy as jnp
from jax import lax
from jax.experimental import pallas as pl
from jax.experimental.pallas import tpu as pltpu
from jax.experimental.pallas import tpu_sc as plsc

N = 10000
E = 320000
D = 128
EPS = 1e-5

NC = 2
NS = 16
NW = NC * NS
CHUNK = 128
NCHT = E // CHUNK
CPW = NCHT // NW
EXTRA = (6, 14, 22, 30)
R0 = 624
RLAST = N - (NS - 1) * R0


def _segsum_body(src_hbm, dst_hbm, z_hbm, zeros_hbm, out_hbm,
                 sa, da, sb, db, rows_a, rows_b, sem_a, sem_b, agg):
    c = lax.axis_index("c")
    s = lax.axis_index("s")
    w = c * NS + s
    nxtra = sum((w > e).astype(jnp.int32) for e in EXTRA)
    nch = CPW + sum((w == e).astype(jnp.int32) for e in EXTRA)
    base = pl.multiple_of((CPW * w + nxtra) * CHUNK, 8)
    row0 = pl.multiple_of(s * R0, 8)

    @pl.when(s < NS - 1)
    def _():
        pltpu.sync_copy(zeros_hbm.at[pl.ds(0, R0)], agg.at[pl.ds(row0, R0)])

    @pl.when(s == NS - 1)
    def _():
        pltpu.sync_copy(zeros_hbm, agg.at[pl.ds((NS - 1) * R0, RLAST)])

    plsc.subcore_barrier()

    def body(i, carry):
        off = base + i * CHUNK
        pltpu.sync_copy(src_hbm.at[pl.ds(off, CHUNK)], sa)
        pltpu.sync_copy(dst_hbm.at[pl.ds(off, CHUNK)], da)
        pltpu.async_copy(z_hbm.at[sa], rows_a, sem_a).wait()
        pltpu.sync_copy(rows_a, agg.at[da], add=True)
        return carry

    lax.fori_loop(0, nch, body, 0)

    plsc.subcore_barrier()
    obase = pl.multiple_of(c * N + row0, 8)

    @pl.when(s < NS - 1)
    def _():
        pltpu.sync_copy(agg.at[pl.ds(row0, R0)], out_hbm.at[pl.ds(obase, R0)])

    @pl.when(s == NS - 1)
    def _():
        pltpu.sync_copy(agg.at[pl.ds((NS - 1) * R0, RLAST)],
                        out_hbm.at[pl.ds(c * N + (NS - 1) * R0, RLAST)])


def _segment_sum(z, src, dst, zeros):
    mesh = plsc.VectorSubcoreMesh(core_axis_name="c", subcore_axis_name="s")
    k = pl.kernel(
        _segsum_body,
        mesh=mesh,
        out_type=jax.ShapeDtypeStruct((2 * N, D), jnp.float32),
        scratch_types=[
            pltpu.VMEM((CHUNK,), jnp.int32),
            pltpu.VMEM((CHUNK,), jnp.int32),
            pltpu.VMEM((CHUNK,), jnp.int32),
            pltpu.VMEM((CHUNK,), jnp.int32),
            pltpu.VMEM((CHUNK, D), jnp.float32),
            pltpu.VMEM((CHUNK, D), jnp.float32),
            pltpu.SemaphoreType.DMA,
            pltpu.SemaphoreType.DMA,
            pltpu.VMEM_SHARED((N, D), jnp.float32),
        ],
    )
    return k(src, dst, z, zeros)


BM = 1000


def _mlp_body(x_ref, p0_ref, p1_ref, w1_ref, b1_ref, w2_ref, b2_ref, o_ref):
    h = x_ref[...] + p0_ref[...] + p1_ref[...]
    h = jnp.dot(h, w1_ref[...], preferred_element_type=jnp.float32) + b1_ref[...]
    h = jnp.maximum(h, 0.0)
    h = jnp.dot(h, w2_ref[...], preferred_element_type=jnp.float32) + b2_ref[...]
    o_ref[...] = jnp.maximum(h, 0.0)


def _gin_mlp(x, parts, w1, b1, w2, b2):
    nb = N // BM
    return pl.pallas_call(
        _mlp_body,
        grid=(nb,),
        in_specs=[
            pl.BlockSpec((BM, D), lambda i: (i, 0)),
            pl.BlockSpec((BM, D), lambda i: (i, 0)),
            pl.BlockSpec((BM, D), lambda i, nb=nb: (i + nb, 0)),
            pl.BlockSpec((D, D), lambda i: (0, 0)),
            pl.BlockSpec((1, D), lambda i: (0, 0)),
            pl.BlockSpec((D, D), lambda i: (0, 0)),
            pl.BlockSpec((1, D), lambda i: (0, 0)),
        ],
        out_specs=pl.BlockSpec((BM, D), lambda i: (i, 0)),
        out_shape=jax.ShapeDtypeStruct((N, D), jnp.float32),
    )(x, parts, parts, w1, b1.reshape(1, D), w2, b2.reshape(1, D))


def _final_body(z2_ref, wp_ref, bp_ref, bng_ref, bnb_ref, png_ref, pnb_ref,
                pw_ref, z_ref, p_ref):
    z2 = z2_ref[...]
    m = jnp.mean(z2, axis=0, keepdims=True)
    v = jnp.mean((z2 - m) ** 2, axis=0, keepdims=True)
    z = (z2 - m) / jnp.sqrt(v + EPS) * bng_ref[...] + bnb_ref[...]
    z_ref[...] = z
    pp = jnp.dot(z, wp_ref[...], preferred_element_type=jnp.float32) + bp_ref[...]
    m2 = jnp.mean(pp, axis=0, keepdims=True)
    v2 = jnp.mean((pp - m2) ** 2, axis=0, keepdims=True)
    p = (pp - m2) / jnp.sqrt(v2 + EPS) * png_ref[...] + pnb_ref[...]
    p_ref[...] = jnp.where(p >= 0.0, p, pw_ref[0, 0] * p)


def _final(z2, wp, bp, bn_g, bn_b, pn_g, pn_b, prelu_w):
    return pl.pallas_call(
        _final_body,
        out_shape=(
            jax.ShapeDtypeStruct((N, D), jnp.float32),
            jax.ShapeDtypeStruct((N, D), jnp.float32),
        ),
    )(z2, wp, bp.reshape(1, D), bn_g.reshape(1, D), bn_b.reshape(1, D),
      pn_g.reshape(1, D), pn_b.reshape(1, D), prelu_w.reshape(1, 1))


def kernel(x, edge_index, W1_0, b1_0, W2_0, b2_0, W1_1, b1_1, W2_1, b2_1,
           bn_g, bn_b, Wp, bp, pn_g, pn_b, prelu_w):
    src = edge_index[0]
    dst = edge_index[1]
    zeros = jnp.zeros((RLAST, D), jnp.float32)
    parts0 = _segment_sum(x, src, dst, zeros)
    z1 = _gin_mlp(x, parts0, W1_0, b1_0, W2_0, b2_0)
    parts1 = _segment_sum(z1, src, dst, zeros)
    z2 = _gin_mlp(z1, parts1, W1_1, b1_1, W2_1, b2_1)
    z, p = _final(z2, Wp, bp, bn_g, bn_b, pn_g, pn_b, prelu_w)
    return (z, p)


# speedup vs baseline: 3.0481x; 1.5251x over previous
import jax
import jax.numpy as jnp
from jax import lax
from jax.experimental import pallas as pl
from jax.experimental.pallas import tpu as pltpu
from jax.experimental.pallas import tpu_sc as plsc

N = 10000
E = 320000
D = 128
EPS = 1e-5

NC = 2
NS = 16
NW = NC * NS
CHUNK = 128
NCHT = E // CHUNK
CPW = NCHT // NW
EXTRA = (6, 14, 22, 30)
R0 = 624
RLAST = N - (NS - 1) * R0


def _segsum_body(src_hbm, dst_hbm, z_hbm, zeros_hbm, out_hbm,
                 sa, da, sb, db, rows_a, rows_b, sem_a, sem_b, agg):
    c = lax.axis_index("c")
    s = lax.axis_index("s")
    w = c * NS + s
    nxtra = sum((w > e).astype(jnp.int32) for e in EXTRA)
    nch = CPW + sum((w == e).astype(jnp.int32) for e in EXTRA)
    base = pl.multiple_of((CPW * w + nxtra) * CHUNK, 8)
    row0 = pl.multiple_of(s * R0, 8)

    @pl.when(s < NS - 1)
    def _():
        pltpu.sync_copy(zeros_hbm.at[pl.ds(0, R0)], agg.at[pl.ds(row0, R0)])

    @pl.when(s == NS - 1)
    def _():
        pltpu.sync_copy(zeros_hbm, agg.at[pl.ds((NS - 1) * R0, RLAST)])

    plsc.subcore_barrier()

    def load_gather(off, sidx, didx, rows, sem):
        pltpu.sync_copy(src_hbm.at[pl.ds(off, CHUNK)], sidx)
        pltpu.sync_copy(dst_hbm.at[pl.ds(off, CHUNK)], didx)
        pltpu.async_copy(z_hbm.at[sidx], rows, sem)

    load_gather(base, sa, da, rows_a, sem_a)

    def body(i, carry):
        off_next = base + (i + 1) * CHUNK

        @pl.when(i % 2 == 0)
        def _():
            @pl.when(i + 1 < nch)
            def _():
                load_gather(off_next, sb, db, rows_b, sem_b)

            pltpu.make_async_copy(z_hbm.at[sa], rows_a, sem_a).wait()
            pltpu.sync_copy(rows_a, agg.at[da], add=True)

        @pl.when(i % 2 == 1)
        def _():
            @pl.when(i + 1 < nch)
            def _():
                load_gather(off_next, sa, da, rows_a, sem_a)

            pltpu.make_async_copy(z_hbm.at[sb], rows_b, sem_b).wait()
            pltpu.sync_copy(rows_b, agg.at[db], add=True)

        return carry

    lax.fori_loop(0, nch, body, 0)

    plsc.subcore_barrier()
    obase = pl.multiple_of(c * N + row0, 8)

    @pl.when(s < NS - 1)
    def _():
        pltpu.sync_copy(agg.at[pl.ds(row0, R0)], out_hbm.at[pl.ds(obase, R0)])

    @pl.when(s == NS - 1)
    def _():
        pltpu.sync_copy(agg.at[pl.ds((NS - 1) * R0, RLAST)],
                        out_hbm.at[pl.ds(c * N + (NS - 1) * R0, RLAST)])


def _segment_sum(z, src, dst, zeros):
    mesh = plsc.VectorSubcoreMesh(core_axis_name="c", subcore_axis_name="s")
    k = pl.kernel(
        _segsum_body,
        mesh=mesh,
        out_type=jax.ShapeDtypeStruct((2 * N, D), jnp.float32),
        scratch_types=[
            pltpu.VMEM((CHUNK,), jnp.int32),
            pltpu.VMEM((CHUNK,), jnp.int32),
            pltpu.VMEM((CHUNK,), jnp.int32),
            pltpu.VMEM((CHUNK,), jnp.int32),
            pltpu.VMEM((CHUNK, D), jnp.float32),
            pltpu.VMEM((CHUNK, D), jnp.float32),
            pltpu.SemaphoreType.DMA,
            pltpu.SemaphoreType.DMA,
            pltpu.VMEM_SHARED((N, D), jnp.float32),
        ],
    )
    return k(src, dst, z, zeros)


BM = 1000


def _mlp_body(x_ref, p0_ref, p1_ref, w1_ref, b1_ref, w2_ref, b2_ref, o_ref):
    h = x_ref[...] + p0_ref[...] + p1_ref[...]
    h = jnp.dot(h, w1_ref[...], preferred_element_type=jnp.float32) + b1_ref[...]
    h = jnp.maximum(h, 0.0)
    h = jnp.dot(h, w2_ref[...], preferred_element_type=jnp.float32) + b2_ref[...]
    o_ref[...] = jnp.maximum(h, 0.0)


def _gin_mlp(x, parts, w1, b1, w2, b2):
    nb = N // BM
    return pl.pallas_call(
        _mlp_body,
        grid=(nb,),
        in_specs=[
            pl.BlockSpec((BM, D), lambda i: (i, 0)),
            pl.BlockSpec((BM, D), lambda i: (i, 0)),
            pl.BlockSpec((BM, D), lambda i, nb=nb: (i + nb, 0)),
            pl.BlockSpec((D, D), lambda i: (0, 0)),
            pl.BlockSpec((1, D), lambda i: (0, 0)),
            pl.BlockSpec((D, D), lambda i: (0, 0)),
            pl.BlockSpec((1, D), lambda i: (0, 0)),
        ],
        out_specs=pl.BlockSpec((BM, D), lambda i: (i, 0)),
        out_shape=jax.ShapeDtypeStruct((N, D), jnp.float32),
    )(x, parts, parts, w1, b1.reshape(1, D), w2, b2.reshape(1, D))


def _final_body(z2_ref, wp_ref, bp_ref, bng_ref, bnb_ref, png_ref, pnb_ref,
                pw_ref, z_ref, p_ref):
    z2 = z2_ref[...]
    m = jnp.mean(z2, axis=0, keepdims=True)
    v = jnp.mean((z2 - m) ** 2, axis=0, keepdims=True)
    z = (z2 - m) / jnp.sqrt(v + EPS) * bng_ref[...] + bnb_ref[...]
    z_ref[...] = z
    pp = jnp.dot(z, wp_ref[...], preferred_element_type=jnp.float32) + bp_ref[...]
    m2 = jnp.mean(pp, axis=0, keepdims=True)
    v2 = jnp.mean((pp - m2) ** 2, axis=0, keepdims=True)
    p = (pp - m2) / jnp.sqrt(v2 + EPS) * png_ref[...] + pnb_ref[...]
    p_ref[...] = jnp.where(p >= 0.0, p, pw_ref[0, 0] * p)


def _final(z2, wp, bp, bn_g, bn_b, pn_g, pn_b, prelu_w):
    return pl.pallas_call(
        _final_body,
        out_shape=(
            jax.ShapeDtypeStruct((N, D), jnp.float32),
            jax.ShapeDtypeStruct((N, D), jnp.float32),
        ),
    )(z2, wp, bp.reshape(1, D), bn_g.reshape(1, D), bn_b.reshape(1, D),
      pn_g.reshape(1, D), pn_b.reshape(1, D), prelu_w.reshape(1, 1))


def kernel(x, edge_index, W1_0, b1_0, W2_0, b2_0, W1_1, b1_1, W2_1, b2_1,
           bn_g, bn_b, Wp, bp, pn_g, pn_b, prelu_w):
    src = edge_index[0]
    dst = edge_index[1]
    zeros = jnp.zeros((RLAST, D), jnp.float32)
    parts0 = _segment_sum(x, src, dst, zeros)
    z1 = _gin_mlp(x, parts0, W1_0, b1_0, W2_0, b2_0)
    parts1 = _segment_sum(z1, src, dst, zeros)
    z2 = _gin_mlp(z1, parts1, W1_1, b1_1, W2_1, b2_1)
    z, p = _final(z2, Wp, bp, bn_g, bn_b, pn_g, pn_b, prelu_w)
    return (z, p)


# speedup vs baseline: 3.6114x; 1.1848x over previous
import jax
import jax.numpy as jnp
from jax import lax
from jax.experimental import pallas as pl
from jax.experimental.pallas import tpu as pltpu
from jax.experimental.pallas import tpu_sc as plsc

N = 10000
E = 320000
D = 128
EPS = 1e-5

NC = 2
NS = 16
NW = NC * NS
CHUNK = 128
NCHT = E // CHUNK
CPW = NCHT // NW
EXTRA = (6, 14, 22, 30)
R0 = 624
RLAST = N - (NS - 1) * R0


def _segsum_body(src_hbm, dst_hbm, z_hbm, zeros_hbm, out_hbm,
                 sa, da, sb, db, sc_, dc, rows_a, rows_b, rows_c,
                 gsem_a, gsem_b, gsem_c, ssem_a, ssem_b, ssem_c, agg):
    c = lax.axis_index("c")
    s = lax.axis_index("s")
    w = c * NS + s
    nxtra = sum((w > e).astype(jnp.int32) for e in EXTRA)
    nch = CPW + sum((w == e).astype(jnp.int32) for e in EXTRA)
    base = pl.multiple_of((CPW * w + nxtra) * CHUNK, 8)
    row0 = pl.multiple_of(s * R0, 8)

    @pl.when(s < NS - 1)
    def _():
        pltpu.sync_copy(zeros_hbm.at[pl.ds(0, R0)], agg.at[pl.ds(row0, R0)])

    @pl.when(s == NS - 1)
    def _():
        pltpu.sync_copy(zeros_hbm, agg.at[pl.ds((NS - 1) * R0, RLAST)])

    plsc.subcore_barrier()

    bufs = ((sa, da, rows_a, gsem_a, ssem_a),
            (sb, db, rows_b, gsem_b, ssem_b),
            (sc_, dc, rows_c, gsem_c, ssem_c))

    def load_gather(off, b):
        sidx, didx, rows, gsem, _ = bufs[b]
        pltpu.sync_copy(src_hbm.at[pl.ds(off, CHUNK)], sidx)
        pltpu.sync_copy(dst_hbm.at[pl.ds(off, CHUNK)], didx)
        pltpu.async_copy(z_hbm.at[sidx], rows, gsem)

    def wait_scatter(b):
        sidx, didx, rows, _, ssem = bufs[b]
        pltpu.make_async_copy(rows, agg.at[didx], ssem).wait()

    load_gather(base, 0)

    def body(i, carry):
        off_next = base + (i + 1) * CHUNK
        for b in range(3):
            @pl.when(i % 3 == b)
            def _(b=b):
                sidx, didx, rows, gsem, ssem = bufs[b]
                nb = (b + 1) % 3

                @pl.when(i + 1 < nch)
                def _():
                    @pl.when(i >= 2)
                    def _():
                        wait_scatter(nb)

                    load_gather(off_next, nb)

                pltpu.make_async_copy(z_hbm.at[sidx], rows, gsem).wait()
                pltpu.async_copy(rows, agg.at[didx], ssem, add=True)

        return carry

    lax.fori_loop(0, nch, body, 0)

    wait_scatter(0)
    wait_scatter(1)
    wait_scatter(2)

    plsc.subcore_barrier()
    obase = pl.multiple_of(c * N + row0, 8)

    @pl.when(s < NS - 1)
    def _():
        pltpu.sync_copy(agg.at[pl.ds(row0, R0)], out_hbm.at[pl.ds(obase, R0)])

    @pl.when(s == NS - 1)
    def _():
        pltpu.sync_copy(agg.at[pl.ds((NS - 1) * R0, RLAST)],
                        out_hbm.at[pl.ds(c * N + (NS - 1) * R0, RLAST)])


def _segment_sum(z, src, dst, zeros):
    mesh = plsc.VectorSubcoreMesh(core_axis_name="c", subcore_axis_name="s")
    k = pl.kernel(
        _segsum_body,
        mesh=mesh,
        out_type=jax.ShapeDtypeStruct((2 * N, D), jnp.float32),
        scratch_types=[
            pltpu.VMEM((CHUNK,), jnp.int32),
            pltpu.VMEM((CHUNK,), jnp.int32),
            pltpu.VMEM((CHUNK,), jnp.int32),
            pltpu.VMEM((CHUNK,), jnp.int32),
            pltpu.VMEM((CHUNK,), jnp.int32),
            pltpu.VMEM((CHUNK,), jnp.int32),
            pltpu.VMEM((CHUNK, D), jnp.float32),
            pltpu.VMEM((CHUNK, D), jnp.float32),
            pltpu.VMEM((CHUNK, D), jnp.float32),
            pltpu.SemaphoreType.DMA,
            pltpu.SemaphoreType.DMA,
            pltpu.SemaphoreType.DMA,
            pltpu.SemaphoreType.DMA,
            pltpu.SemaphoreType.DMA,
            pltpu.SemaphoreType.DMA,
            pltpu.VMEM_SHARED((N, D), jnp.float32),
        ],
    )
    return k(src, dst, z, zeros)


BM = 1000


def _mlp_body(x_ref, p0_ref, p1_ref, w1_ref, b1_ref, w2_ref, b2_ref, o_ref):
    h = x_ref[...] + p0_ref[...] + p1_ref[...]
    h = jnp.dot(h, w1_ref[...], preferred_element_type=jnp.float32) + b1_ref[...]
    h = jnp.maximum(h, 0.0)
    h = jnp.dot(h, w2_ref[...], preferred_element_type=jnp.float32) + b2_ref[...]
    o_ref[...] = jnp.maximum(h, 0.0)


def _gin_mlp(x, parts, w1, b1, w2, b2):
    nb = N // BM
    return pl.pallas_call(
        _mlp_body,
        grid=(nb,),
        in_specs=[
            pl.BlockSpec((BM, D), lambda i: (i, 0)),
            pl.BlockSpec((BM, D), lambda i: (i, 0)),
            pl.BlockSpec((BM, D), lambda i, nb=nb: (i + nb, 0)),
            pl.BlockSpec((D, D), lambda i: (0, 0)),
            pl.BlockSpec((1, D), lambda i: (0, 0)),
            pl.BlockSpec((D, D), lambda i: (0, 0)),
            pl.BlockSpec((1, D), lambda i: (0, 0)),
        ],
        out_specs=pl.BlockSpec((BM, D), lambda i: (i, 0)),
        out_shape=jax.ShapeDtypeStruct((N, D), jnp.float32),
    )(x, parts, parts, w1, b1.reshape(1, D), w2, b2.reshape(1, D))


def _final_body(z2_ref, wp_ref, bp_ref, bng_ref, bnb_ref, png_ref, pnb_ref,
                pw_ref, z_ref, p_ref):
    z2 = z2_ref[...]
    m = jnp.mean(z2, axis=0, keepdims=True)
    v = jnp.mean((z2 - m) ** 2, axis=0, keepdims=True)
    z = (z2 - m) / jnp.sqrt(v + EPS) * bng_ref[...] + bnb_ref[...]
    z_ref[...] = z
    pp = jnp.dot(z, wp_ref[...], preferred_element_type=jnp.float32) + bp_ref[...]
    m2 = jnp.mean(pp, axis=0, keepdims=True)
    v2 = jnp.mean((pp - m2) ** 2, axis=0, keepdims=True)
    p = (pp - m2) / jnp.sqrt(v2 + EPS) * png_ref[...] + pnb_ref[...]
    p_ref[...] = jnp.where(p >= 0.0, p, pw_ref[0, 0] * p)


def _final(z2, wp, bp, bn_g, bn_b, pn_g, pn_b, prelu_w):
    return pl.pallas_call(
        _final_body,
        out_shape=(
            jax.ShapeDtypeStruct((N, D), jnp.float32),
            jax.ShapeDtypeStruct((N, D), jnp.float32),
        ),
    )(z2, wp, bp.reshape(1, D), bn_g.reshape(1, D), bn_b.reshape(1, D),
      pn_g.reshape(1, D), pn_b.reshape(1, D), prelu_w.reshape(1, 1))


def kernel(x, edge_index, W1_0, b1_0, W2_0, b2_0, W1_1, b1_1, W2_1, b2_1,
           bn_g, bn_b, Wp, bp, pn_g, pn_b, prelu_w):
    src = edge_index[0]
    dst = edge_index[1]
    zeros = jnp.zeros((RLAST, D), jnp.float32)
    parts0 = _segment_sum(x, src, dst, zeros)
    z1 = _gin_mlp(x, parts0, W1_0, b1_0, W2_0, b2_0)
    parts1 = _segment_sum(z1, src, dst, zeros)
    z2 = _gin_mlp(z1, parts1, W1_1, b1_1, W2_1, b2_1)
    z, p = _final(z2, Wp, bp, bn_g, bn_b, pn_g, pn_b, prelu_w)
    return (z, p)


# speedup vs baseline: 3.6550x; 1.0121x over previous
import jax
import jax.numpy as jnp
from jax import lax
from jax.experimental import pallas as pl
from jax.experimental.pallas import tpu as pltpu
from jax.experimental.pallas import tpu_sc as plsc

N = 10000
E = 320000
D = 128
EPS = 1e-5

NC = 2
NS = 16
NW = NC * NS
CHUNK = 128
NCHT = E // CHUNK
CPW = NCHT // NW
EXTRA = (6, 14, 22, 30)
R0 = 624
RLAST = N - (NS - 1) * R0


def _segsum_body(src_hbm, dst_hbm, z_hbm, zeros_hbm, out_hbm,
                 sa, da, sb, db, sc_, dc, rows_a, rows_b, rows_c,
                 gsem_a, gsem_b, gsem_c, ssem_a, ssem_b, ssem_c, agg):
    c = lax.axis_index("c")
    s = lax.axis_index("s")
    w = c * NS + s
    nxtra = sum((w > e).astype(jnp.int32) for e in EXTRA)
    nch = CPW + sum((w == e).astype(jnp.int32) for e in EXTRA)
    base = pl.multiple_of((CPW * w + nxtra) * CHUNK, 8)
    row0 = pl.multiple_of(s * R0, 8)

    @pl.when(s < NS - 1)
    def _():
        pltpu.sync_copy(zeros_hbm.at[pl.ds(0, R0)], agg.at[pl.ds(row0, R0)])

    @pl.when(s == NS - 1)
    def _():
        pltpu.sync_copy(zeros_hbm, agg.at[pl.ds((NS - 1) * R0, RLAST)])

    plsc.subcore_barrier()

    bufs = ((sa, da, rows_a, gsem_a, ssem_a),
            (sb, db, rows_b, gsem_b, ssem_b),
            (sc_, dc, rows_c, gsem_c, ssem_c))

    def load_gather(off, b):
        sidx, didx, rows, gsem, _ = bufs[b]
        pltpu.sync_copy(src_hbm.at[pl.ds(off, CHUNK)], sidx)
        pltpu.sync_copy(dst_hbm.at[pl.ds(off, CHUNK)], didx)
        pltpu.async_copy(z_hbm.at[sidx], rows, gsem)

    def wait_scatter(b):
        sidx, didx, rows, _, ssem = bufs[b]
        pltpu.make_async_copy(rows, agg.at[didx], ssem).wait()

    load_gather(base, 0)

    def body(i, carry):
        off_next = base + (i + 1) * CHUNK
        for b in range(3):
            @pl.when(i % 3 == b)
            def _(b=b):
                sidx, didx, rows, gsem, ssem = bufs[b]
                nb = (b + 1) % 3

                @pl.when(i + 1 < nch)
                def _():
                    @pl.when(i >= 2)
                    def _():
                        wait_scatter(nb)

                    load_gather(off_next, nb)

                pltpu.make_async_copy(z_hbm.at[sidx], rows, gsem).wait()
                pltpu.async_copy(rows, agg.at[didx], ssem, add=True)

        return carry

    lax.fori_loop(0, nch, body, 0)

    wait_scatter(0)
    wait_scatter(1)
    wait_scatter(2)

    plsc.subcore_barrier()
    obase = pl.multiple_of(c * N + row0, 8)

    @pl.when(s < NS - 1)
    def _():
        pltpu.sync_copy(agg.at[pl.ds(row0, R0)], out_hbm.at[pl.ds(obase, R0)])

    @pl.when(s == NS - 1)
    def _():
        pltpu.sync_copy(agg.at[pl.ds((NS - 1) * R0, RLAST)],
                        out_hbm.at[pl.ds(c * N + (NS - 1) * R0, RLAST)])


def _segment_sum(z, src, dst, zeros):
    mesh = plsc.VectorSubcoreMesh(core_axis_name="c", subcore_axis_name="s")
    k = pl.kernel(
        _segsum_body,
        mesh=mesh,
        out_type=jax.ShapeDtypeStruct((2 * N, D), jnp.float32),
        scratch_types=[
            pltpu.VMEM((CHUNK,), jnp.int32),
            pltpu.VMEM((CHUNK,), jnp.int32),
            pltpu.VMEM((CHUNK,), jnp.int32),
            pltpu.VMEM((CHUNK,), jnp.int32),
            pltpu.VMEM((CHUNK,), jnp.int32),
            pltpu.VMEM((CHUNK,), jnp.int32),
            pltpu.VMEM((CHUNK, D), jnp.float32),
            pltpu.VMEM((CHUNK, D), jnp.float32),
            pltpu.VMEM((CHUNK, D), jnp.float32),
            pltpu.SemaphoreType.DMA,
            pltpu.SemaphoreType.DMA,
            pltpu.SemaphoreType.DMA,
            pltpu.SemaphoreType.DMA,
            pltpu.SemaphoreType.DMA,
            pltpu.SemaphoreType.DMA,
            pltpu.VMEM_SHARED((N, D), jnp.float32),
        ],
    )
    return k(src, dst, z, zeros)


BM = 1000


def _mlp_body(x_ref, p0_ref, p1_ref, w1_ref, b1_ref, w2_ref, b2_ref, o_ref):
    h = x_ref[...] + p0_ref[...] + p1_ref[...]
    h = jnp.dot(h, w1_ref[...], preferred_element_type=jnp.float32) + b1_ref[...]
    h = jnp.maximum(h, 0.0)
    h = jnp.dot(h, w2_ref[...], preferred_element_type=jnp.float32) + b2_ref[...]
    o_ref[...] = jnp.maximum(h, 0.0)


def _gin_mlp(x, parts, w1, b1, w2, b2):
    nb = N // BM
    return pl.pallas_call(
        _mlp_body,
        grid=(nb,),
        in_specs=[
            pl.BlockSpec((BM, D), lambda i: (i, 0)),
            pl.BlockSpec((BM, D), lambda i: (i, 0)),
            pl.BlockSpec((BM, D), lambda i, nb=nb: (i + nb, 0)),
            pl.BlockSpec((D, D), lambda i: (0, 0)),
            pl.BlockSpec((1, D), lambda i: (0, 0)),
            pl.BlockSpec((D, D), lambda i: (0, 0)),
            pl.BlockSpec((1, D), lambda i: (0, 0)),
        ],
        out_specs=pl.BlockSpec((BM, D), lambda i: (i, 0)),
        out_shape=jax.ShapeDtypeStruct((N, D), jnp.float32),
    )(x, parts, parts, w1, b1.reshape(1, D), w2, b2.reshape(1, D))


def _final_body(z1_ref, p0_ref, p1_ref, w1_ref, b1_ref, w2_ref, b2_ref,
                wp_ref, bp_ref, bng_ref, bnb_ref, png_ref, pnb_ref,
                pw_ref, z_ref, p_ref):
    h = z1_ref[...] + p0_ref[...] + p1_ref[...]
    h = jnp.dot(h, w1_ref[...], preferred_element_type=jnp.float32) + b1_ref[...]
    h = jnp.maximum(h, 0.0)
    h = jnp.dot(h, w2_ref[...], preferred_element_type=jnp.float32) + b2_ref[...]
    z2 = jnp.maximum(h, 0.0)
    m = jnp.mean(z2, axis=0, keepdims=True)
    v = jnp.mean((z2 - m) ** 2, axis=0, keepdims=True)
    z = (z2 - m) / jnp.sqrt(v + EPS) * bng_ref[...] + bnb_ref[...]
    z_ref[...] = z
    pp = jnp.dot(z, wp_ref[...], preferred_element_type=jnp.float32) + bp_ref[...]
    m2 = jnp.mean(pp, axis=0, keepdims=True)
    v2 = jnp.mean((pp - m2) ** 2, axis=0, keepdims=True)
    p = (pp - m2) / jnp.sqrt(v2 + EPS) * png_ref[...] + pnb_ref[...]
    p_ref[...] = jnp.where(p >= 0.0, p, pw_ref[0, 0] * p)


def _final(z1, parts, w1, b1, w2, b2, wp, bp, bn_g, bn_b, pn_g, pn_b,
           prelu_w):
    p0 = parts[:N]
    p1 = parts[N:]
    return pl.pallas_call(
        _final_body,
        out_shape=(
            jax.ShapeDtypeStruct((N, D), jnp.float32),
            jax.ShapeDtypeStruct((N, D), jnp.float32),
        ),
    )(z1, p0, p1, w1, b1.reshape(1, D), w2, b2.reshape(1, D),
      wp, bp.reshape(1, D), bn_g.reshape(1, D), bn_b.reshape(1, D),
      pn_g.reshape(1, D), pn_b.reshape(1, D), prelu_w.reshape(1, 1))


def kernel(x, edge_index, W1_0, b1_0, W2_0, b2_0, W1_1, b1_1, W2_1, b2_1,
           bn_g, bn_b, Wp, bp, pn_g, pn_b, prelu_w):
    src = edge_index[0]
    dst = edge_index[1]
    zeros = jnp.zeros((RLAST, D), jnp.float32)
    parts0 = _segment_sum(x, src, dst, zeros)
    z1 = _gin_mlp(x, parts0, W1_0, b1_0, W2_0, b2_0)
    parts1 = _segment_sum(z1, src, dst, zeros)
    z, p = _final(z1, parts1, W1_1, b1_1, W2_1, b2_1, Wp, bp,
                  bn_g, bn_b, pn_g, pn_b, prelu_w)
    return (z, p)


# speedup vs baseline: 4.2239x; 1.1556x over previous
import jax
import jax.numpy as jnp
from jax import lax
from jax.experimental import pallas as pl
from jax.experimental.pallas import tpu as pltpu
from jax.experimental.pallas import tpu_sc as plsc

N = 10000
E = 320000
D = 128
EPS = 1e-5

NC = 2
NS = 16
NW = NC * NS
CHUNK = 128
NCHT = E // CHUNK
CPW = NCHT // NW
EXTRA = (6, 14, 22, 30)
R0 = 624
RLAST = N - (NS - 1) * R0


NIB = 6


def _segsum_body(src_hbm, dst_hbm, z_hbm, zeros_hbm, out_hbm,
                 s0, d0, s1, d1, s2, d2, s3, d3, s4, d4, s5, d5,
                 rows_a, rows_b, rows_c,
                 is0, id0, is1, id1, is2, id2, is3, id3, is4, id4, is5, id5,
                 gsem_a, gsem_b, gsem_c, ssem_a, ssem_b, ssem_c, agg):
    c = lax.axis_index("c")
    s = lax.axis_index("s")
    w = c * NS + s
    nxtra = sum((w > e).astype(jnp.int32) for e in EXTRA)
    nch = CPW + sum((w == e).astype(jnp.int32) for e in EXTRA)
    base = pl.multiple_of((CPW * w + nxtra) * CHUNK, 8)
    row0 = pl.multiple_of(s * R0, 8)

    @pl.when(s < NS - 1)
    def _():
        pltpu.sync_copy(zeros_hbm.at[pl.ds(0, R0)], agg.at[pl.ds(row0, R0)])

    @pl.when(s == NS - 1)
    def _():
        pltpu.sync_copy(zeros_hbm, agg.at[pl.ds((NS - 1) * R0, RLAST)])

    plsc.subcore_barrier()

    rbufs = ((rows_a, gsem_a, ssem_a),
             (rows_b, gsem_b, ssem_b),
             (rows_c, gsem_c, ssem_c))
    ibufs = ((s0, d0, is0, id0), (s1, d1, is1, id1), (s2, d2, is2, id2),
             (s3, d3, is3, id3), (s4, d4, is4, id4), (s5, d5, is5, id5))

    def idx_load(off, p):
        sidx, didx, isem, idsem = ibufs[p]
        pltpu.async_copy(src_hbm.at[pl.ds(off, CHUNK)], sidx, isem)
        pltpu.async_copy(dst_hbm.at[pl.ds(off, CHUNK)], didx, idsem)

    def idx_wait(off, p):
        sidx, didx, isem, idsem = ibufs[p]
        pltpu.make_async_copy(src_hbm.at[pl.ds(off, CHUNK)], sidx, isem).wait()
        pltpu.make_async_copy(dst_hbm.at[pl.ds(off, CHUNK)], didx, idsem).wait()

    def gather(p, b):
        sidx = ibufs[p][0]
        rows, gsem, _ = rbufs[b]
        pltpu.async_copy(z_hbm.at[sidx], rows, gsem)

    def wait_scatter(p, b):
        didx = ibufs[p][1]
        rows, _, ssem = rbufs[b]
        pltpu.make_async_copy(rows, agg.at[didx], ssem).wait()

    idx_load(base, 0)
    idx_load(base + CHUNK, 1)
    idx_wait(base, 0)
    gather(0, 0)

    def body(i, carry):
        off1 = base + (i + 1) * CHUNK
        off2 = base + (i + 2) * CHUNK
        for p in range(NIB):
            @pl.when(i % NIB == p)
            def _(p=p):
                b = p % 3
                np_, nb = (p + 1) % NIB, (p + 1) % 3
                pp = (p + 2) % NIB
                prev_p = (p + 4) % NIB
                rows, gsem, _ = rbufs[b]
                didx = ibufs[p][1]
                ssem = rbufs[b][2]

                @pl.when(i + 1 < nch)
                def _():
                    @pl.when(i >= 2)
                    def _():
                        wait_scatter(prev_p, nb)

                    idx_wait(off1, np_)
                    gather(np_, nb)

                @pl.when(i + 2 < nch)
                def _():
                    idx_load(off2, pp)

                pltpu.make_async_copy(z_hbm.at[ibufs[p][0]], rows, gsem).wait()
                pltpu.async_copy(rows, agg.at[didx], ssem, add=True)

        return carry

    lax.fori_loop(0, nch, body, 0)

    for k in range(3):
        ch = nch - 3 + k
        for p in range(NIB):
            @pl.when(ch % NIB == p)
            def _(p=p):
                wait_scatter(p, p % 3)

    plsc.subcore_barrier()
    obase = pl.multiple_of(c * N + row0, 8)

    @pl.when(s < NS - 1)
    def _():
        pltpu.sync_copy(agg.at[pl.ds(row0, R0)], out_hbm.at[pl.ds(obase, R0)])

    @pl.when(s == NS - 1)
    def _():
        pltpu.sync_copy(agg.at[pl.ds((NS - 1) * R0, RLAST)],
                        out_hbm.at[pl.ds(c * N + (NS - 1) * R0, RLAST)])


def _segment_sum(z, src, dst, zeros):
    mesh = plsc.VectorSubcoreMesh(core_axis_name="c", subcore_axis_name="s")
    k = pl.kernel(
        _segsum_body,
        mesh=mesh,
        out_type=jax.ShapeDtypeStruct((2 * N, D), jnp.float32),
        scratch_types=(
            [pltpu.VMEM((CHUNK,), jnp.int32)] * (2 * NIB)
            + [pltpu.VMEM((CHUNK, D), jnp.float32)] * 3
            + [pltpu.SemaphoreType.DMA] * (2 * NIB)
            + [pltpu.SemaphoreType.DMA] * 6
            + [pltpu.VMEM_SHARED((N, D), jnp.float32)]
        ),
    )
    return k(src, dst, z, zeros)


BM = 1000


def _mlp_body(x_ref, p0_ref, p1_ref, w1_ref, b1_ref, w2_ref, b2_ref, o_ref):
    h = x_ref[...] + p0_ref[...] + p1_ref[...]
    h = jnp.dot(h, w1_ref[...], preferred_element_type=jnp.float32) + b1_ref[...]
    h = jnp.maximum(h, 0.0)
    h = jnp.dot(h, w2_ref[...], preferred_element_type=jnp.float32) + b2_ref[...]
    o_ref[...] = jnp.maximum(h, 0.0)


def _gin_mlp(x, parts, w1, b1, w2, b2):
    nb = N // BM
    return pl.pallas_call(
        _mlp_body,
        grid=(nb,),
        in_specs=[
            pl.BlockSpec((BM, D), lambda i: (i, 0)),
            pl.BlockSpec((BM, D), lambda i: (i, 0)),
            pl.BlockSpec((BM, D), lambda i, nb=nb: (i + nb, 0)),
            pl.BlockSpec((D, D), lambda i: (0, 0)),
            pl.BlockSpec((1, D), lambda i: (0, 0)),
            pl.BlockSpec((D, D), lambda i: (0, 0)),
            pl.BlockSpec((1, D), lambda i: (0, 0)),
        ],
        out_specs=pl.BlockSpec((BM, D), lambda i: (i, 0)),
        out_shape=jax.ShapeDtypeStruct((N, D), jnp.float32),
    )(x, parts, parts, w1, b1.reshape(1, D), w2, b2.reshape(1, D))


def _final_body(z1_ref, p0_ref, p1_ref, w1_ref, b1_ref, w2_ref, b2_ref,
                wp_ref, bp_ref, bng_ref, bnb_ref, png_ref, pnb_ref,
                pw_ref, z_ref, p_ref):
    h = z1_ref[...] + p0_ref[...] + p1_ref[...]
    h = jnp.dot(h, w1_ref[...], preferred_element_type=jnp.float32) + b1_ref[...]
    h = jnp.maximum(h, 0.0)
    h = jnp.dot(h, w2_ref[...], preferred_element_type=jnp.float32) + b2_ref[...]
    z2 = jnp.maximum(h, 0.0)
    m = jnp.mean(z2, axis=0, keepdims=True)
    v = jnp.mean((z2 - m) ** 2, axis=0, keepdims=True)
    z = (z2 - m) / jnp.sqrt(v + EPS) * bng_ref[...] + bnb_ref[...]
    z_ref[...] = z
    pp = jnp.dot(z, wp_ref[...], preferred_element_type=jnp.float32) + bp_ref[...]
    m2 = jnp.mean(pp, axis=0, keepdims=True)
    v2 = jnp.mean((pp - m2) ** 2, axis=0, keepdims=True)
    p = (pp - m2) / jnp.sqrt(v2 + EPS) * png_ref[...] + pnb_ref[...]
    p_ref[...] = jnp.where(p >= 0.0, p, pw_ref[0, 0] * p)


def _final(z1, parts, w1, b1, w2, b2, wp, bp, bn_g, bn_b, pn_g, pn_b,
           prelu_w):
    p0 = parts[:N]
    p1 = parts[N:]
    return pl.pallas_call(
        _final_body,
        out_shape=(
            jax.ShapeDtypeStruct((N, D), jnp.float32),
            jax.ShapeDtypeStruct((N, D), jnp.float32),
        ),
    )(z1, p0, p1, w1, b1.reshape(1, D), w2, b2.reshape(1, D),
      wp, bp.reshape(1, D), bn_g.reshape(1, D), bn_b.reshape(1, D),
      pn_g.reshape(1, D), pn_b.reshape(1, D), prelu_w.reshape(1, 1))


def kernel(x, edge_index, W1_0, b1_0, W2_0, b2_0, W1_1, b1_1, W2_1, b2_1,
           bn_g, bn_b, Wp, bp, pn_g, pn_b, prelu_w):
    src = edge_index[0]
    dst = edge_index[1]
    zeros = jnp.zeros((RLAST, D), jnp.float32)
    parts0 = _segment_sum(x, src, dst, zeros)
    z1 = _gin_mlp(x, parts0, W1_0, b1_0, W2_0, b2_0)
    parts1 = _segment_sum(z1, src, dst, zeros)
    z, p = _final(z1, parts1, W1_1, b1_1, W2_1, b2_1, Wp, bp,
                  bn_g, bn_b, pn_g, pn_b, prelu_w)
    return (z, p)
